# baseline (speedup 1.0000x reference)
import functools

import jax
import jax.numpy as jnp
import numpy as np
from jax import lax
from jax.experimental import pallas as pl
from jax.experimental.pallas import tpu as pltpu
from jax.experimental.pallas import tpu_sc as plsc

NC, NS, LANES = 2, 16, 16
NW = NC * NS
CH = 32
NBUF = 4


def _mm_bias_body(a_ref, w_ref, b_ref, o_ref):
    o_ref[...] = (
        jnp.dot(a_ref[...], w_ref[...], preferred_element_type=jnp.float32)
        + b_ref[...]
    ).astype(o_ref.dtype)


def _mm_body(a_ref, w_ref, o_ref):
    o_ref[...] = jnp.dot(a_ref[...], w_ref[...], preferred_element_type=jnp.float32)


def _tc_matmul_bias(a, w, b, bn, out_dtype=jnp.float32):
    m, k = a.shape
    n = w.shape[1]
    return pl.pallas_call(
        _mm_bias_body,
        grid=(m // bn,),
        in_specs=[
            pl.BlockSpec((bn, k), lambda i: (i, 0)),
            pl.BlockSpec((k, n), lambda i: (0, 0)),
            pl.BlockSpec((1, n), lambda i: (0, 0)),
        ],
        out_specs=pl.BlockSpec((bn, n), lambda i: (i, 0)),
        out_shape=jax.ShapeDtypeStruct((m, n), out_dtype),
    )(a, w, b)


def _tc_matmul(a, w, bn):
    m, k = a.shape
    n = w.shape[1]
    return pl.pallas_call(
        _mm_body,
        grid=(m // bn,),
        in_specs=[
            pl.BlockSpec((bn, k), lambda i: (i, 0)),
            pl.BlockSpec((k, n), lambda i: (0, 0)),
        ],
        out_specs=pl.BlockSpec((bn, n), lambda i: (i, 0)),
        out_shape=jax.ShapeDtypeStruct((m, n), jnp.float32),
    )(a, w)


def _final_body(x_ref, u0_ref, u1_ref, v0_ref, v1_ref,
                wu1_ref, bu1_ref, wu2_ref, bu2_ref,
                wb1_ref, bb1_ref, wb2_ref, bb2_ref,
                wc0_ref, wc1_ref, bc_ref, scale_ref, o_ref):
    scale = scale_ref[0, 0]
    xb = x_ref[...]
    h_up = u0_ref[...] + u1_ref[...] + scale * xb
    t = jnp.maximum(
        jnp.dot(h_up, wu1_ref[...], preferred_element_type=jnp.float32)
        + bu1_ref[...], 0.0)
    out_up = jnp.maximum(
        jnp.dot(t, wu2_ref[...], preferred_element_type=jnp.float32)
        + bu2_ref[...], 0.0)
    h_b = v0_ref[...] + v1_ref[...] + scale * xb
    t2 = jnp.maximum(
        jnp.dot(h_b, wb1_ref[...], preferred_element_type=jnp.float32)
        + bb1_ref[...], 0.0)
    out_b = jnp.maximum(
        jnp.dot(t2, wb2_ref[...], preferred_element_type=jnp.float32)
        + bb2_ref[...], 0.0)
    o_ref[...] = jnp.maximum(
        jnp.dot(out_up, wc0_ref[...], preferred_element_type=jnp.float32)
        + jnp.dot(out_b, wc1_ref[...], preferred_element_type=jnp.float32)
        + bc_ref[...], 0.0)


def _tc_final(x, u0, u1, v0, v1, wu1, bu1, wu2, bu2,
              wb1, bb1, wb2, bb2, wc0, wc1, bc, scale, bn):
    n_rows, d = x.shape
    mat = lambda: pl.BlockSpec((d, d), lambda i: (0, 0))
    vec = lambda: pl.BlockSpec((1, d), lambda i: (0, 0))
    rows = lambda: pl.BlockSpec((bn, d), lambda i: (i, 0))
    return pl.pallas_call(
        _final_body,
        grid=(n_rows // bn,),
        in_specs=[
            rows(), rows(), rows(), rows(), rows(),
            mat(), vec(), mat(), vec(),
            mat(), vec(), mat(), vec(),
            mat(), mat(), vec(),
            pl.BlockSpec(memory_space=pltpu.SMEM),
        ],
        out_specs=pl.BlockSpec((bn, d), lambda i: (i, 0)),
        out_shape=jax.ShapeDtypeStruct((n_rows, d), jnp.float32),
    )(x, u0, u1, v0, v1, wu1, bu1, wu2, bu2,
      wb1, bb1, wb2, bb2, wc0, wc1, bc, scale)


def _make_sc_pass(d, n_pairs, nt_rows, rps, with_ua):
    ppw = n_pairs // NW
    full = ppw // CH
    tail_n = ppw - full * CH
    assert full % NBUF == 0

    mesh = plsc.VectorSubcoreMesh(core_axis_name="c", subcore_axis_name="s")

    scratch = [
        pltpu.VMEM_SHARED((nt_rows, d), jnp.float32),
        pltpu.VMEM((NBUF, CH, d), jnp.float32),
        pltpu.VMEM((NBUF, CH), jnp.int32),
        pltpu.VMEM((NBUF, CH), jnp.int32),
        pltpu.SemaphoreType.DMA((NBUF,)),
        pltpu.SemaphoreType.DMA((NBUF,)),
        pltpu.SemaphoreType.DMA((NBUF,)),
        pltpu.SemaphoreType.DMA,
    ]
    if with_ua:
        scratch += [
            pltpu.VMEM((NBUF, CH, d // 2), jnp.int32),
            pltpu.SemaphoreType.DMA((NBUF,)),
        ]
    if tail_n:
        scratch += [
            pltpu.VMEM((tail_n, d), jnp.float32),
            pltpu.VMEM((tail_n,), jnp.int32),
            pltpu.VMEM((tail_n,), jnp.int32),
        ]
        if with_ua:
            scratch += [pltpu.VMEM((tail_n, d // 2), jnp.int32)]

    @functools.partial(
        pl.kernel,
        out_type=[jax.ShapeDtypeStruct((nt_rows, d), jnp.float32)] * 2,
        mesh=mesh,
        scratch_types=scratch,
    )
    def sc_pass(*refs):
        it = iter(refs)
        tbl_hbm = next(it)
        ua_hbm = next(it) if with_ua else None
        src_hbm = next(it)
        dst_hbm = next(it)
        z_hbm = next(it)
        o0_hbm = next(it)
        o1_hbm = next(it)
        acc = next(it)
        rows_v = next(it)
        si = next(it)
        di = next(it)
        sem_i = next(it)
        sem_g = next(it)
        sem_s = next(it)
        sem_t = next(it)
        if with_ua:
            ua_v = next(it)
            sem_u = next(it)
        if tail_n:
            rows_t = next(it)
            si_t = next(it)
            di_t = next(it)
            if with_ua:
                ua_t = next(it)

        c = lax.axis_index("c")
        s = lax.axis_index("s")
        wid = s * NC + c
        row0 = s * rps
        base = wid * ppw
        nfull = full

        def relu_add_row(rows, ua, r):
            for k in range(d // 32):
                u = ua[r, pl.ds(16 * k, LANES)]
                lo = lax.bitcast_convert_type(u << 16, jnp.float32)
                hi = lax.bitcast_convert_type(u & jnp.int32(-65536), jnp.float32)
                sl0 = pl.ds(32 * k, LANES)
                sl1 = pl.ds(32 * k + LANES, LANES)
                rows[r, sl0] = jnp.maximum(rows[r, sl0] + lo, 0.0)
                rows[r, sl1] = jnp.maximum(rows[r, sl1] + hi, 0.0)

        def relu_add(b):
            def row_fn(r2, carry):
                for dr in range(2):
                    relu_add_row(rows_v.at[b], ua_v.at[b], r2 * 2 + dr)
                return carry
            lax.fori_loop(0, CH // 2, row_fn, 0)

        def pipeline():
            def fire_idx(g, b):
                off = base + g * CH
                pltpu.async_copy(src_hbm.at[pl.ds(off, CH)], si.at[b],
                                 sem_i.at[b])
                pltpu.async_copy(dst_hbm.at[pl.ds(off, CH)], di.at[b],
                                 sem_i.at[b])
                if with_ua:
                    pltpu.async_copy(ua_hbm.at[pl.ds(off, CH)], ua_v.at[b],
                                     sem_u.at[b])

            def wait_idx(b):
                pltpu.make_async_copy(src_hbm.at[pl.ds(0, CH)], si.at[b],
                                      sem_i.at[b]).wait()
                pltpu.make_async_copy(dst_hbm.at[pl.ds(0, CH)], di.at[b],
                                      sem_i.at[b]).wait()

            def fire_gather(b):
                pltpu.async_copy(tbl_hbm.at[si.at[b]], rows_v.at[b],
                                 sem_g.at[b])

            def wait_gather(b):
                pltpu.make_async_copy(tbl_hbm.at[si.at[b]], rows_v.at[b],
                                      sem_g.at[b]).wait()

            def wait_ua(b):
                pltpu.make_async_copy(ua_hbm.at[pl.ds(0, CH)], ua_v.at[b],
                                      sem_u.at[b]).wait()

            def fire_scatter(b):
                pltpu.async_copy(rows_v.at[b], acc.at[di.at[b]], sem_s.at[b],
                                 add=True)

            def wait_scatter(b):
                pltpu.make_async_copy(rows_v.at[b], acc.at[di.at[b]],
                                      sem_s.at[b]).wait()

            fire_idx(0, 0)
            fire_idx(1, 1)
            wait_idx(0)
            fire_gather(0)

            def group(i, carry):
                g0 = i * NBUF
                for db in range(NBUF):
                    g = g0 + db
                    b2 = (db + 2) % NBUF

                    @pl.when(jnp.logical_and(g + 2 >= NBUF, g + 2 < nfull))
                    def _():
                        wait_scatter(b2)

                    @pl.when(g + 2 < nfull)
                    def _():
                        fire_idx(g + 2, b2)

                    b1 = (db + 1) % NBUF

                    @pl.when(g + 1 < nfull)
                    def _():
                        wait_idx(b1)
                        fire_gather(b1)

                    if with_ua:
                        wait_ua(db)
                    wait_gather(db)
                    if with_ua:
                        relu_add(db)
                    fire_scatter(db)
                return carry

            lax.fori_loop(0, nfull // NBUF, group, 0)
            for b in range(NBUF):
                wait_scatter(b)

        pltpu.sync_copy(z_hbm, acc.at[pl.ds(row0, rps)])
        plsc.subcore_barrier()

        pipeline()

        if tail_n:
            off = base + full * CH
            pltpu.sync_copy(src_hbm.at[pl.ds(off, tail_n)], si_t)
            pltpu.sync_copy(dst_hbm.at[pl.ds(off, tail_n)], di_t)
            if with_ua:
                pltpu.sync_copy(ua_hbm.at[pl.ds(off, tail_n)], ua_t)
            pltpu.async_copy(tbl_hbm.at[si_t], rows_t, sem_t).wait()

            if with_ua:
                def trow(r, carry):
                    relu_add_row(rows_t, ua_t, r)
                    return carry
                lax.fori_loop(0, tail_n, trow, 0)
            pltpu.sync_copy(rows_t, acc.at[di_t], add=True)

        plsc.subcore_barrier()

        @pl.when(c == 0)
        def _():
            pltpu.sync_copy(acc.at[pl.ds(row0, rps)], o0_hbm.at[pl.ds(row0, rps)])

        @pl.when(c == 1)
        def _():
            pltpu.sync_copy(acc.at[pl.ds(row0, rps)], o1_hbm.at[pl.ds(row0, rps)])

    return sc_pass


def kernel(x, up_index, up_attr, boundary_attr, boundary_index,
           W_msg_up, b_msg_up, W_up1, b_up1, W_up2, b_up2,
           W_b1, b_b1, W_b2, b_b2, W_comb, b_comb, eps1):
    n_cells, d = x.shape
    e_edges = up_attr.shape[0]
    eb = boundary_index.shape[1]

    rps = -(-(n_cells + 1) // NS)
    rps = -(-rps // 8) * 8
    nt_rows = rps * NS
    eb_pad = -(-eb // (NW * CH * NBUF)) * (NW * CH * NBUF)

    w_top = W_msg_up[:d]
    w_bot = W_msg_up[d:]

    src = up_index[0]
    dst = up_index[1]
    pad = eb_pad - eb
    bj = jnp.concatenate([boundary_index[0], jnp.zeros((pad,), jnp.int32)])
    bi = jnp.concatenate([boundary_index[1],
                          jnp.full((pad,), n_cells, jnp.int32)])
    zrows = jnp.zeros((rps, d), jnp.float32)

    sc_bnd = _make_sc_pass(d, eb_pad, nt_rows, rps, with_ua=False)
    b0, b1 = sc_bnd(boundary_attr, bj, bi, zrows)

    nu = np.arange(d).reshape(d // 32, 2, 16).transpose(0, 2, 1).reshape(-1)
    ua_bf = _tc_matmul_bias(up_attr, w_bot[:, nu], b_msg_up[nu].reshape(1, d),
                            bn=2000, out_dtype=jnp.bfloat16)
    ua = lax.bitcast_convert_type(
        ua_bf.reshape(e_edges, d // 2, 2), jnp.int32)
    xp = _tc_matmul(x, w_top, bn=1000)

    sc_edge = _make_sc_pass(d, e_edges, nt_rows, rps, with_ua=True)
    up0, up1 = sc_edge(xp, ua, src, dst, zrows)

    scale = (1.0 + eps1).reshape(1, 1)
    out = _tc_final(x, up0, up1, b0, b1,
                    W_up1, b_up1.reshape(1, d), W_up2, b_up2.reshape(1, d),
                    W_b1, b_b1.reshape(1, d), W_b2, b_b2.reshape(1, d),
                    W_comb[:d], W_comb[d:], b_comb.reshape(1, d),
                    scale, bn=1000)
    return out


# speedup vs baseline: 2.2548x; 2.2548x over previous
import functools

import jax
import jax.numpy as jnp
import numpy as np
from jax import lax
from jax.experimental import pallas as pl
from jax.experimental.pallas import tpu as pltpu
from jax.experimental.pallas import tpu_sc as plsc

NC, NS, LANES = 2, 16, 16
NW = NC * NS
CH = 32
NBUF = 4


def _mm_bias_body(a_ref, w_ref, b_ref, o_ref):
    o_ref[...] = (
        jnp.dot(a_ref[...], w_ref[...], preferred_element_type=jnp.float32)
        + b_ref[...]
    ).astype(o_ref.dtype)


def _mm_bias_pack_body(a_ref, w_ref, b_ref, o_ref):
    v = (jnp.dot(a_ref[...], w_ref[...], preferred_element_type=jnp.float32)
         + b_ref[...])
    h = v.shape[1] // 2
    lo = lax.bitcast_convert_type(v[:, :h].astype(jnp.bfloat16), jnp.uint16)
    hi = lax.bitcast_convert_type(v[:, h:].astype(jnp.bfloat16), jnp.uint16)
    o_ref[...] = lo.astype(jnp.int32) | (hi.astype(jnp.int32) << 16)


def _mm_body(a_ref, w_ref, o_ref):
    o_ref[...] = jnp.dot(a_ref[...], w_ref[...], preferred_element_type=jnp.float32)


def _tc_matmul_bias(a, w, b, bn, pack_u32=False):
    m, k = a.shape
    n = w.shape[1]
    body = _mm_bias_pack_body if pack_u32 else _mm_bias_body
    on = n // 2 if pack_u32 else n
    odt = jnp.int32 if pack_u32 else jnp.float32
    return pl.pallas_call(
        body,
        grid=(m // bn,),
        in_specs=[
            pl.BlockSpec((bn, k), lambda i: (i, 0)),
            pl.BlockSpec((k, n), lambda i: (0, 0)),
            pl.BlockSpec((1, n), lambda i: (0, 0)),
        ],
        out_specs=pl.BlockSpec((bn, on), lambda i: (i, 0)),
        out_shape=jax.ShapeDtypeStruct((m, on), odt),
    )(a, w, b)


def _tc_matmul(a, w, bn):
    m, k = a.shape
    n = w.shape[1]
    return pl.pallas_call(
        _mm_body,
        grid=(m // bn,),
        in_specs=[
            pl.BlockSpec((bn, k), lambda i: (i, 0)),
            pl.BlockSpec((k, n), lambda i: (0, 0)),
        ],
        out_specs=pl.BlockSpec((bn, n), lambda i: (i, 0)),
        out_shape=jax.ShapeDtypeStruct((m, n), jnp.float32),
    )(a, w)


def _final_body(x_ref, u0_ref, u1_ref, v0_ref, v1_ref,
                wu1_ref, bu1_ref, wu2_ref, bu2_ref,
                wb1_ref, bb1_ref, wb2_ref, bb2_ref,
                wc0_ref, wc1_ref, bc_ref, scale_ref, o_ref):
    scale = scale_ref[0, 0]
    xb = x_ref[...]
    h_up = u0_ref[...] + u1_ref[...] + scale * xb
    t = jnp.maximum(
        jnp.dot(h_up, wu1_ref[...], preferred_element_type=jnp.float32)
        + bu1_ref[...], 0.0)
    out_up = jnp.maximum(
        jnp.dot(t, wu2_ref[...], preferred_element_type=jnp.float32)
        + bu2_ref[...], 0.0)
    h_b = v0_ref[...] + v1_ref[...] + scale * xb
    t2 = jnp.maximum(
        jnp.dot(h_b, wb1_ref[...], preferred_element_type=jnp.float32)
        + bb1_ref[...], 0.0)
    out_b = jnp.maximum(
        jnp.dot(t2, wb2_ref[...], preferred_element_type=jnp.float32)
        + bb2_ref[...], 0.0)
    o_ref[...] = jnp.maximum(
        jnp.dot(out_up, wc0_ref[...], preferred_element_type=jnp.float32)
        + jnp.dot(out_b, wc1_ref[...], preferred_element_type=jnp.float32)
        + bc_ref[...], 0.0)


def _tc_final(x, u0, u1, v0, v1, wu1, bu1, wu2, bu2,
              wb1, bb1, wb2, bb2, wc0, wc1, bc, scale, bn):
    n_rows, d = x.shape
    mat = lambda: pl.BlockSpec((d, d), lambda i: (0, 0))
    vec = lambda: pl.BlockSpec((1, d), lambda i: (0, 0))
    rows = lambda: pl.BlockSpec((bn, d), lambda i: (i, 0))
    return pl.pallas_call(
        _final_body,
        grid=(n_rows // bn,),
        in_specs=[
            rows(), rows(), rows(), rows(), rows(),
            mat(), vec(), mat(), vec(),
            mat(), vec(), mat(), vec(),
            mat(), mat(), vec(),
            pl.BlockSpec(memory_space=pltpu.SMEM),
        ],
        out_specs=pl.BlockSpec((bn, d), lambda i: (i, 0)),
        out_shape=jax.ShapeDtypeStruct((n_rows, d), jnp.float32),
    )(x, u0, u1, v0, v1, wu1, bu1, wu2, bu2,
      wb1, bb1, wb2, bb2, wc0, wc1, bc, scale)


def _make_sc_pass(d, n_pairs, nt_rows, rps, with_ua):
    ppw = n_pairs // NW
    full = ppw // CH
    tail_n = ppw - full * CH
    assert full % NBUF == 0

    mesh = plsc.VectorSubcoreMesh(core_axis_name="c", subcore_axis_name="s")

    scratch = [
        pltpu.VMEM_SHARED((nt_rows, d), jnp.float32),
        pltpu.VMEM((NBUF, CH, d), jnp.float32),
        pltpu.VMEM((NBUF, CH), jnp.int32),
        pltpu.VMEM((NBUF, CH), jnp.int32),
        pltpu.SemaphoreType.DMA((NBUF,)),
        pltpu.SemaphoreType.DMA((NBUF,)),
        pltpu.SemaphoreType.DMA((NBUF,)),
        pltpu.SemaphoreType.DMA,
    ]
    if with_ua:
        scratch += [
            pltpu.VMEM((NBUF, CH, d // 2), jnp.int32),
            pltpu.SemaphoreType.DMA((NBUF,)),
        ]
    if tail_n:
        scratch += [
            pltpu.VMEM((tail_n, d), jnp.float32),
            pltpu.VMEM((tail_n,), jnp.int32),
            pltpu.VMEM((tail_n,), jnp.int32),
        ]
        if with_ua:
            scratch += [pltpu.VMEM((tail_n, d // 2), jnp.int32)]

    @functools.partial(
        pl.kernel,
        out_type=[jax.ShapeDtypeStruct((nt_rows, d), jnp.float32)] * 2,
        mesh=mesh,
        scratch_types=scratch,
    )
    def sc_pass(*refs):
        it = iter(refs)
        tbl_hbm = next(it)
        ua_hbm = next(it) if with_ua else None
        src_hbm = next(it)
        dst_hbm = next(it)
        z_hbm = next(it)
        o0_hbm = next(it)
        o1_hbm = next(it)
        acc = next(it)
        rows_v = next(it)
        si = next(it)
        di = next(it)
        sem_i = next(it)
        sem_g = next(it)
        sem_s = next(it)
        sem_t = next(it)
        if with_ua:
            ua_v = next(it)
            sem_u = next(it)
        if tail_n:
            rows_t = next(it)
            si_t = next(it)
            di_t = next(it)
            if with_ua:
                ua_t = next(it)

        c = lax.axis_index("c")
        s = lax.axis_index("s")
        wid = s * NC + c
        row0 = s * rps
        base = wid * ppw
        nfull = full

        def relu_add_row(rows, ua, r):
            for k in range(d // 32):
                u = ua[r, pl.ds(16 * k, LANES)]
                lo = lax.bitcast_convert_type(u << 16, jnp.float32)
                hi = lax.bitcast_convert_type(u & jnp.int32(-65536), jnp.float32)
                sl0 = pl.ds(16 * k, LANES)
                sl1 = pl.ds(d // 2 + 16 * k, LANES)
                rows[r, sl0] = jnp.maximum(rows[r, sl0] + lo, 0.0)
                rows[r, sl1] = jnp.maximum(rows[r, sl1] + hi, 0.0)

        def relu_add(b):
            def row_fn(r2, carry):
                for dr in range(2):
                    relu_add_row(rows_v.at[b], ua_v.at[b], r2 * 2 + dr)
                return carry
            lax.fori_loop(0, CH // 2, row_fn, 0)

        def pipeline():
            def fire_idx(g, b):
                off = base + g * CH
                pltpu.async_copy(src_hbm.at[pl.ds(off, CH)], si.at[b],
                                 sem_i.at[b])
                pltpu.async_copy(dst_hbm.at[pl.ds(off, CH)], di.at[b],
                                 sem_i.at[b])
                if with_ua:
                    pltpu.async_copy(ua_hbm.at[pl.ds(off, CH)], ua_v.at[b],
                                     sem_u.at[b])

            def wait_idx(b):
                pltpu.make_async_copy(src_hbm.at[pl.ds(0, CH)], si.at[b],
                                      sem_i.at[b]).wait()
                pltpu.make_async_copy(dst_hbm.at[pl.ds(0, CH)], di.at[b],
                                      sem_i.at[b]).wait()

            def fire_gather(b):
                pltpu.async_copy(tbl_hbm.at[si.at[b]], rows_v.at[b],
                                 sem_g.at[b])

            def wait_gather(b):
                pltpu.make_async_copy(tbl_hbm.at[si.at[b]], rows_v.at[b],
                                      sem_g.at[b]).wait()

            def wait_ua(b):
                pltpu.make_async_copy(ua_hbm.at[pl.ds(0, CH)], ua_v.at[b],
                                      sem_u.at[b]).wait()

            def fire_scatter(b):
                pltpu.async_copy(rows_v.at[b], acc.at[di.at[b]], sem_s.at[b],
                                 add=True)

            def wait_scatter(b):
                pltpu.make_async_copy(rows_v.at[b], acc.at[di.at[b]],
                                      sem_s.at[b]).wait()

            fire_idx(0, 0)
            fire_idx(1, 1)
            wait_idx(0)
            fire_gather(0)

            def group(i, carry):
                g0 = i * NBUF
                for db in range(NBUF):
                    g = g0 + db
                    b2 = (db + 2) % NBUF

                    @pl.when(jnp.logical_and(g + 2 >= NBUF, g + 2 < nfull))
                    def _():
                        wait_scatter(b2)

                    @pl.when(g + 2 < nfull)
                    def _():
                        fire_idx(g + 2, b2)

                    b1 = (db + 1) % NBUF

                    @pl.when(g + 1 < nfull)
                    def _():
                        wait_idx(b1)
                        fire_gather(b1)

                    if with_ua:
                        wait_ua(db)
                    wait_gather(db)
                    if with_ua:
                        relu_add(db)
                    fire_scatter(db)
                return carry

            lax.fori_loop(0, nfull // NBUF, group, 0)
            for b in range(NBUF):
                wait_scatter(b)

        pltpu.sync_copy(z_hbm, acc.at[pl.ds(row0, rps)])
        plsc.subcore_barrier()

        pipeline()

        if tail_n:
            off = base + full * CH
            pltpu.sync_copy(src_hbm.at[pl.ds(off, tail_n)], si_t)
            pltpu.sync_copy(dst_hbm.at[pl.ds(off, tail_n)], di_t)
            if with_ua:
                pltpu.sync_copy(ua_hbm.at[pl.ds(off, tail_n)], ua_t)
            pltpu.async_copy(tbl_hbm.at[si_t], rows_t, sem_t).wait()

            if with_ua:
                def trow(r, carry):
                    relu_add_row(rows_t, ua_t, r)
                    return carry
                lax.fori_loop(0, tail_n, trow, 0)
            pltpu.sync_copy(rows_t, acc.at[di_t], add=True)

        plsc.subcore_barrier()

        @pl.when(c == 0)
        def _():
            pltpu.sync_copy(acc.at[pl.ds(row0, rps)], o0_hbm.at[pl.ds(row0, rps)])

        @pl.when(c == 1)
        def _():
            pltpu.sync_copy(acc.at[pl.ds(row0, rps)], o1_hbm.at[pl.ds(row0, rps)])

    return sc_pass


def kernel(x, up_index, up_attr, boundary_attr, boundary_index,
           W_msg_up, b_msg_up, W_up1, b_up1, W_up2, b_up2,
           W_b1, b_b1, W_b2, b_b2, W_comb, b_comb, eps1):
    n_cells, d = x.shape
    e_edges = up_attr.shape[0]
    eb = boundary_index.shape[1]

    rps = -(-(n_cells + 1) // NS)
    rps = -(-rps // 8) * 8
    nt_rows = rps * NS
    eb_pad = -(-eb // (NW * CH * NBUF)) * (NW * CH * NBUF)

    w_top = W_msg_up[:d]
    w_bot = W_msg_up[d:]

    src = up_index[0]
    dst = up_index[1]
    pad = eb_pad - eb
    bj = jnp.concatenate([boundary_index[0], jnp.zeros((pad,), jnp.int32)])
    bi = jnp.concatenate([boundary_index[1],
                          jnp.full((pad,), n_cells, jnp.int32)])
    zrows = jnp.zeros((rps, d), jnp.float32)

    sc_bnd = _make_sc_pass(d, eb_pad, nt_rows, rps, with_ua=False)
    b0, b1 = sc_bnd(boundary_attr, bj, bi, zrows)

    ua = _tc_matmul_bias(up_attr, w_bot, b_msg_up.reshape(1, d),
                         bn=2000, pack_u32=True)
    xp = _tc_matmul(x, w_top, bn=1000)

    sc_edge = _make_sc_pass(d, e_edges, nt_rows, rps, with_ua=True)
    up0, up1 = sc_edge(xp, ua, src, dst, zrows)

    scale = (1.0 + eps1).reshape(1, 1)
    out = _tc_final(x, up0, up1, b0, b1,
                    W_up1, b_up1.reshape(1, d), W_up2, b_up2.reshape(1, d),
                    W_b1, b_b1.reshape(1, d), W_b2, b_b2.reshape(1, d),
                    W_comb[:d], W_comb[d:], b_comb.reshape(1, d),
                    scale, bn=1000)
    return out


# speedup vs baseline: 2.5617x; 1.1361x over previous
import functools

import jax
import jax.numpy as jnp
import numpy as np
from jax import lax
from jax.experimental import pallas as pl
from jax.experimental.pallas import tpu as pltpu
from jax.experimental.pallas import tpu_sc as plsc

NC, NS, LANES = 2, 16, 16
NW = NC * NS
CH = 32
NBUF = 4


def _mm_bias_body(a_ref, w_ref, b_ref, o_ref):
    o_ref[...] = (
        jnp.dot(a_ref[...], w_ref[...], preferred_element_type=jnp.float32)
        + b_ref[...]
    ).astype(o_ref.dtype)


def _mm_bias_pack_body(a_ref, w_ref, b_ref, o_ref):
    v = (jnp.dot(a_ref[...], w_ref[...], preferred_element_type=jnp.float32)
         + b_ref[...])
    h = v.shape[1] // 2
    lo = lax.bitcast_convert_type(v[:, :h].astype(jnp.bfloat16), jnp.uint16)
    hi = lax.bitcast_convert_type(v[:, h:].astype(jnp.bfloat16), jnp.uint16)
    o_ref[...] = lo.astype(jnp.int32) | (hi.astype(jnp.int32) << 16)


def _mm_body(a_ref, w_ref, o_ref):
    o_ref[...] = jnp.dot(a_ref[...], w_ref[...], preferred_element_type=jnp.float32)


def _tc_matmul_bias(a, w, b, bn, pack_u32=False):
    m, k = a.shape
    n = w.shape[1]
    body = _mm_bias_pack_body if pack_u32 else _mm_bias_body
    on = n // 2 if pack_u32 else n
    odt = jnp.int32 if pack_u32 else jnp.float32
    return pl.pallas_call(
        body,
        grid=(m // bn,),
        in_specs=[
            pl.BlockSpec((bn, k), lambda i: (i, 0)),
            pl.BlockSpec((k, n), lambda i: (0, 0)),
            pl.BlockSpec((1, n), lambda i: (0, 0)),
        ],
        out_specs=pl.BlockSpec((bn, on), lambda i: (i, 0)),
        out_shape=jax.ShapeDtypeStruct((m, on), odt),
    )(a, w, b)


def _tc_matmul(a, w, bn):
    m, k = a.shape
    n = w.shape[1]
    return pl.pallas_call(
        _mm_body,
        grid=(m // bn,),
        in_specs=[
            pl.BlockSpec((bn, k), lambda i: (i, 0)),
            pl.BlockSpec((k, n), lambda i: (0, 0)),
        ],
        out_specs=pl.BlockSpec((bn, n), lambda i: (i, 0)),
        out_shape=jax.ShapeDtypeStruct((m, n), jnp.float32),
    )(a, w)


def _final_body(x_ref, u0_ref, u1_ref, v0_ref, v1_ref,
                wu1_ref, bu1_ref, wu2_ref, bu2_ref,
                wb1_ref, bb1_ref, wb2_ref, bb2_ref,
                wc0_ref, wc1_ref, bc_ref, scale_ref, o_ref):
    scale = scale_ref[0, 0]
    xb = x_ref[...]
    h_up = u0_ref[...] + u1_ref[...] + scale * xb
    t = jnp.maximum(
        jnp.dot(h_up, wu1_ref[...], preferred_element_type=jnp.float32)
        + bu1_ref[...], 0.0)
    out_up = jnp.maximum(
        jnp.dot(t, wu2_ref[...], preferred_element_type=jnp.float32)
        + bu2_ref[...], 0.0)
    h_b = v0_ref[...] + v1_ref[...] + scale * xb
    t2 = jnp.maximum(
        jnp.dot(h_b, wb1_ref[...], preferred_element_type=jnp.float32)
        + bb1_ref[...], 0.0)
    out_b = jnp.maximum(
        jnp.dot(t2, wb2_ref[...], preferred_element_type=jnp.float32)
        + bb2_ref[...], 0.0)
    o_ref[...] = jnp.maximum(
        jnp.dot(out_up, wc0_ref[...], preferred_element_type=jnp.float32)
        + jnp.dot(out_b, wc1_ref[...], preferred_element_type=jnp.float32)
        + bc_ref[...], 0.0)


def _tc_final(x, u0, u1, v0, v1, wu1, bu1, wu2, bu2,
              wb1, bb1, wb2, bb2, wc0, wc1, bc, scale, bn):
    n_rows, d = x.shape
    mat = lambda: pl.BlockSpec((d, d), lambda i: (0, 0))
    vec = lambda: pl.BlockSpec((1, d), lambda i: (0, 0))
    rows = lambda: pl.BlockSpec((bn, d), lambda i: (i, 0))
    return pl.pallas_call(
        _final_body,
        grid=(n_rows // bn,),
        in_specs=[
            rows(), rows(), rows(), rows(), rows(),
            mat(), vec(), mat(), vec(),
            mat(), vec(), mat(), vec(),
            mat(), mat(), vec(),
            pl.BlockSpec(memory_space=pltpu.SMEM),
        ],
        out_specs=pl.BlockSpec((bn, d), lambda i: (i, 0)),
        out_shape=jax.ShapeDtypeStruct((n_rows, d), jnp.float32),
    )(x, u0, u1, v0, v1, wu1, bu1, wu2, bu2,
      wb1, bb1, wb2, bb2, wc0, wc1, bc, scale)


def _make_sc_pass(d, n_pairs, nt_rows, rps, with_ua):
    ppw = n_pairs // NW
    full = ppw // CH
    tail_n = ppw - full * CH
    assert full % NBUF == 0

    mesh = plsc.VectorSubcoreMesh(core_axis_name="c", subcore_axis_name="s")

    scratch = [
        pltpu.VMEM_SHARED((nt_rows, d), jnp.float32),
        pltpu.VMEM((NBUF, CH, d), jnp.float32),
        pltpu.VMEM((NBUF, CH), jnp.int32),
        pltpu.VMEM((NBUF, CH), jnp.int32),
        pltpu.SemaphoreType.DMA((NBUF,)),
        pltpu.SemaphoreType.DMA((NBUF,)),
        pltpu.SemaphoreType.DMA((NBUF,)),
        pltpu.SemaphoreType.DMA,
    ]
    if with_ua:
        scratch += [
            pltpu.VMEM((NBUF, CH, d // 2), jnp.int32),
            pltpu.SemaphoreType.DMA((NBUF,)),
        ]
    if tail_n:
        scratch += [
            pltpu.VMEM((tail_n, d), jnp.float32),
            pltpu.VMEM((tail_n,), jnp.int32),
            pltpu.VMEM((tail_n,), jnp.int32),
        ]
        if with_ua:
            scratch += [pltpu.VMEM((tail_n, d // 2), jnp.int32)]

    @functools.partial(
        pl.kernel,
        out_type=[jax.ShapeDtypeStruct((nt_rows, d), jnp.float32)] * 2,
        mesh=mesh,
        scratch_types=scratch,
    )
    def sc_pass(*refs):
        it = iter(refs)
        tbl_hbm = next(it)
        if with_ua:
            ua_hbm = next(it)
            next(it)
        src_hbm = next(it)
        dst_hbm = next(it)
        z_hbm = next(it)
        o0_hbm = next(it)
        o1_hbm = next(it)
        acc = next(it)
        rows_v = next(it)
        si = next(it)
        di = next(it)
        sem_i = next(it)
        sem_g = next(it)
        sem_s = next(it)
        sem_t = next(it)
        if with_ua:
            ua_v = next(it)
            sem_u = next(it)
        if tail_n:
            rows_t = next(it)
            si_t = next(it)
            di_t = next(it)
            if with_ua:
                ua_t = next(it)

        c = lax.axis_index("c")
        s = lax.axis_index("s")
        wid = s * NC + c
        row0 = s * rps
        base = wid * ppw
        nfull = full

        def relu_add_row(rows, ua, r):
            for k in range(d // 32):
                u = ua[r, pl.ds(16 * k, LANES)]
                lo = lax.bitcast_convert_type(u << 16, jnp.float32)
                hi = lax.bitcast_convert_type(u & jnp.int32(-65536), jnp.float32)
                sl0 = pl.ds(16 * k, LANES)
                sl1 = pl.ds(d // 2 + 16 * k, LANES)
                rows[r, sl0] = jnp.maximum(rows[r, sl0] + lo, 0.0)
                rows[r, sl1] = jnp.maximum(rows[r, sl1] + hi, 0.0)

        def relu_add(b):
            def row_fn(r2, carry):
                for dr in range(2):
                    relu_add_row(rows_v.at[b], ua_v.at[b], r2 * 2 + dr)
                return carry
            lax.fori_loop(0, CH // 2, row_fn, 0)

        def pipeline():
            def fire_idx(g, b):
                off = base + g * CH
                pltpu.async_copy(src_hbm.at[pl.ds(off, CH)], si.at[b],
                                 sem_i.at[b])
                pltpu.async_copy(dst_hbm.at[pl.ds(off, CH)], di.at[b],
                                 sem_i.at[b])
                if with_ua:
                    pltpu.async_copy(ua_hbm.at[pl.ds(off, CH)], ua_v.at[b],
                                     sem_u.at[b])

            def wait_idx(b):
                pltpu.make_async_copy(src_hbm.at[pl.ds(0, CH)], si.at[b],
                                      sem_i.at[b]).wait()
                pltpu.make_async_copy(dst_hbm.at[pl.ds(0, CH)], di.at[b],
                                      sem_i.at[b]).wait()

            def fire_gather(b):
                pltpu.async_copy(tbl_hbm.at[si.at[b]], rows_v.at[b],
                                 sem_g.at[b])

            def wait_gather(b):
                pltpu.make_async_copy(tbl_hbm.at[si.at[b]], rows_v.at[b],
                                      sem_g.at[b]).wait()

            def wait_ua(b):
                pltpu.make_async_copy(ua_hbm.at[pl.ds(0, CH)], ua_v.at[b],
                                      sem_u.at[b]).wait()

            def fire_scatter(b):
                pltpu.async_copy(rows_v.at[b], acc.at[di.at[b]], sem_s.at[b],
                                 add=True)

            def wait_scatter(b):
                pltpu.make_async_copy(rows_v.at[b], acc.at[di.at[b]],
                                      sem_s.at[b]).wait()

            fire_idx(0, 0)
            fire_idx(1, 1)
            wait_idx(0)
            fire_gather(0)

            def group(i, carry):
                g0 = i * NBUF
                for db in range(NBUF):
                    g = g0 + db
                    b2 = (db + 2) % NBUF

                    @pl.when(jnp.logical_and(g + 2 >= NBUF, g + 2 < nfull))
                    def _():
                        wait_scatter(b2)

                    @pl.when(g + 2 < nfull)
                    def _():
                        fire_idx(g + 2, b2)

                    b1 = (db + 1) % NBUF

                    @pl.when(g + 1 < nfull)
                    def _():
                        wait_idx(b1)
                        fire_gather(b1)

                    if with_ua:
                        wait_ua(db)
                    wait_gather(db)
                    if with_ua:
                        relu_add(db)
                    fire_scatter(db)
                return carry

            lax.fori_loop(0, nfull // NBUF, group, 0)
            for b in range(NBUF):
                wait_scatter(b)

        pltpu.sync_copy(z_hbm, acc.at[pl.ds(row0, rps)])
        plsc.subcore_barrier()

        pipeline()

        if tail_n:
            off = base + full * CH
            pltpu.sync_copy(src_hbm.at[pl.ds(off, tail_n)], si_t)
            pltpu.sync_copy(dst_hbm.at[pl.ds(off, tail_n)], di_t)
            if with_ua:
                pltpu.sync_copy(ua_hbm.at[pl.ds(off, tail_n)], ua_t)
            pltpu.async_copy(tbl_hbm.at[si_t], rows_t, sem_t).wait()

            if with_ua:
                def trow(r, carry):
                    relu_add_row(rows_t, ua_t, r)
                    return carry
                lax.fori_loop(0, tail_n, trow, 0)
            pltpu.sync_copy(rows_t, acc.at[di_t], add=True)

        plsc.subcore_barrier()

        @pl.when(c == 0)
        def _():
            pltpu.sync_copy(acc.at[pl.ds(row0, rps)], o0_hbm.at[pl.ds(row0, rps)])

        @pl.when(c == 1)
        def _():
            pltpu.sync_copy(acc.at[pl.ds(row0, rps)], o1_hbm.at[pl.ds(row0, rps)])

    return sc_pass


def kernel(x, up_index, up_attr, boundary_attr, boundary_index,
           W_msg_up, b_msg_up, W_up1, b_up1, W_up2, b_up2,
           W_b1, b_b1, W_b2, b_b2, W_comb, b_comb, eps1):
    n_cells, d = x.shape
    e_edges = up_attr.shape[0]
    eb = boundary_index.shape[1]

    rps = -(-(n_cells + 1) // NS)
    rps = -(-rps // 8) * 8
    nt_rows = rps * NS
    eb_pad = -(-eb // (NW * CH * NBUF)) * (NW * CH * NBUF)

    w_top = W_msg_up[:d]
    w_bot = W_msg_up[d:]

    src = up_index[0]
    dst = up_index[1]
    pad = eb_pad - eb
    bj = jnp.concatenate([boundary_index[0], jnp.zeros((pad,), jnp.int32)])
    trash = n_cells + (jnp.arange(pad, dtype=jnp.int32)
                       % jnp.int32(nt_rows - n_cells))
    bi = jnp.concatenate([boundary_index[1], trash])
    zrows = jnp.zeros((rps, d), jnp.float32)

    sc_bnd = _make_sc_pass(d, eb_pad, nt_rows, rps, with_ua=False)
    b0, b1 = sc_bnd(boundary_attr, bj, bi, zrows)

    ua = _tc_matmul_bias(up_attr, w_bot, b_msg_up.reshape(1, d),
                         bn=2000, pack_u32=True)
    xp = _tc_matmul(x, w_top, bn=1000)

    sc_edge = _make_sc_pass(d, e_edges, nt_rows, rps, with_ua=True)
    up0, up1 = sc_edge(xp, ua, b0, src, dst, zrows)

    scale = (1.0 + eps1).reshape(1, 1)
    out = _tc_final(x, up0, up1, b0, b1,
                    W_up1, b_up1.reshape(1, d), W_up2, b_up2.reshape(1, d),
                    W_b1, b_b1.reshape(1, d), W_b2, b_b2.reshape(1, d),
                    W_comb[:d], W_comb[d:], b_comb.reshape(1, d),
                    scale, bn=1000)
    return out


# speedup vs baseline: 2.7182x; 1.0611x over previous
import functools

import jax
import jax.numpy as jnp
import numpy as np
from jax import lax
from jax.experimental import pallas as pl
from jax.experimental.pallas import tpu as pltpu
from jax.experimental.pallas import tpu_sc as plsc

NC, NS, LANES = 2, 16, 16
NW = NC * NS
CH = 32
NBUF = 4


def _mm_bias_body(a_ref, w_ref, b_ref, o_ref):
    o_ref[...] = (
        jnp.dot(a_ref[...], w_ref[...], preferred_element_type=jnp.float32)
        + b_ref[...]
    ).astype(o_ref.dtype)


def _mm_bias_pack_body(a_ref, w_ref, b_ref, o_ref):
    v = (jnp.dot(a_ref[...], w_ref[...], preferred_element_type=jnp.float32)
         + b_ref[...])
    h = v.shape[1] // 2
    lo = lax.bitcast_convert_type(v[:, :h].astype(jnp.bfloat16), jnp.uint16)
    hi = lax.bitcast_convert_type(v[:, h:].astype(jnp.bfloat16), jnp.uint16)
    o_ref[...] = lo.astype(jnp.int32) | (hi.astype(jnp.int32) << 16)


def _mm_body(a_ref, w_ref, o_ref):
    o_ref[...] = jnp.dot(a_ref[...], w_ref[...], preferred_element_type=jnp.float32)


def _tc_matmul_bias(a, w, b, bn, pack_u32=False, rows_m=None, row_off=0):
    m, k = a.shape
    rows_m = m if rows_m is None else rows_m
    off_blocks = row_off // bn
    n = w.shape[1]
    body = _mm_bias_pack_body if pack_u32 else _mm_bias_body
    on = n // 2 if pack_u32 else n
    odt = jnp.int32 if pack_u32 else jnp.float32
    return pl.pallas_call(
        body,
        grid=(rows_m // bn,),
        in_specs=[
            pl.BlockSpec((bn, k), lambda i, o=off_blocks: (i + o, 0)),
            pl.BlockSpec((k, n), lambda i: (0, 0)),
            pl.BlockSpec((1, n), lambda i: (0, 0)),
        ],
        out_specs=pl.BlockSpec((bn, on), lambda i: (i, 0)),
        out_shape=jax.ShapeDtypeStruct((rows_m, on), odt),
    )(a, w, b)


def _tc_matmul(a, w, bn):
    m, k = a.shape
    n = w.shape[1]
    return pl.pallas_call(
        _mm_body,
        grid=(m // bn,),
        in_specs=[
            pl.BlockSpec((bn, k), lambda i: (i, 0)),
            pl.BlockSpec((k, n), lambda i: (0, 0)),
        ],
        out_specs=pl.BlockSpec((bn, n), lambda i: (i, 0)),
        out_shape=jax.ShapeDtypeStruct((m, n), jnp.float32),
    )(a, w)


def _final_body(x_ref, u0_ref, u1_ref, u2_ref, u3_ref, v0_ref, v1_ref,
                wu1_ref, bu1_ref, wu2_ref, bu2_ref,
                wb1_ref, bb1_ref, wb2_ref, bb2_ref,
                wc0_ref, wc1_ref, bc_ref, scale_ref, o_ref):
    scale = scale_ref[0, 0]
    xb = x_ref[...]
    h_up = (u0_ref[...] + u1_ref[...] + u2_ref[...] + u3_ref[...]
            + scale * xb)
    t = jnp.maximum(
        jnp.dot(h_up, wu1_ref[...], preferred_element_type=jnp.float32)
        + bu1_ref[...], 0.0)
    out_up = jnp.maximum(
        jnp.dot(t, wu2_ref[...], preferred_element_type=jnp.float32)
        + bu2_ref[...], 0.0)
    h_b = v0_ref[...] + v1_ref[...] + scale * xb
    t2 = jnp.maximum(
        jnp.dot(h_b, wb1_ref[...], preferred_element_type=jnp.float32)
        + bb1_ref[...], 0.0)
    out_b = jnp.maximum(
        jnp.dot(t2, wb2_ref[...], preferred_element_type=jnp.float32)
        + bb2_ref[...], 0.0)
    o_ref[...] = jnp.maximum(
        jnp.dot(out_up, wc0_ref[...], preferred_element_type=jnp.float32)
        + jnp.dot(out_b, wc1_ref[...], preferred_element_type=jnp.float32)
        + bc_ref[...], 0.0)


def _tc_final(x, u0, u1, u2, u3, v0, v1, wu1, bu1, wu2, bu2,
              wb1, bb1, wb2, bb2, wc0, wc1, bc, scale, bn):
    n_rows, d = x.shape
    mat = lambda: pl.BlockSpec((d, d), lambda i: (0, 0))
    vec = lambda: pl.BlockSpec((1, d), lambda i: (0, 0))
    rows = lambda: pl.BlockSpec((bn, d), lambda i: (i, 0))
    return pl.pallas_call(
        _final_body,
        grid=(n_rows // bn,),
        in_specs=[
            rows(), rows(), rows(), rows(), rows(), rows(), rows(),
            mat(), vec(), mat(), vec(),
            mat(), vec(), mat(), vec(),
            mat(), mat(), vec(),
            pl.BlockSpec(memory_space=pltpu.SMEM),
        ],
        out_specs=pl.BlockSpec((bn, d), lambda i: (i, 0)),
        out_shape=jax.ShapeDtypeStruct((n_rows, d), jnp.float32),
    )(x, u0, u1, u2, u3, v0, v1, wu1, bu1, wu2, bu2,
      wb1, bb1, wb2, bb2, wc0, wc1, bc, scale)


def _make_sc_pass(d, n_pairs, nt_rows, rps, with_ua, e_off=0):
    ppw = n_pairs // NW
    full = ppw // CH
    tail_n = ppw - full * CH
    assert full % NBUF == 0

    mesh = plsc.VectorSubcoreMesh(core_axis_name="c", subcore_axis_name="s")

    scratch = [
        pltpu.VMEM_SHARED((nt_rows, d), jnp.float32),
        pltpu.VMEM((NBUF, CH, d), jnp.float32),
        pltpu.VMEM((NBUF, CH), jnp.int32),
        pltpu.VMEM((NBUF, CH), jnp.int32),
        pltpu.SemaphoreType.DMA((NBUF,)),
        pltpu.SemaphoreType.DMA((NBUF,)),
        pltpu.SemaphoreType.DMA((NBUF,)),
        pltpu.SemaphoreType.DMA,
    ]
    if with_ua:
        scratch += [
            pltpu.VMEM((NBUF, CH, d // 2), jnp.int32),
            pltpu.SemaphoreType.DMA((NBUF,)),
        ]
    if tail_n:
        scratch += [
            pltpu.VMEM((tail_n, d), jnp.float32),
            pltpu.VMEM((tail_n,), jnp.int32),
            pltpu.VMEM((tail_n,), jnp.int32),
        ]
        if with_ua:
            scratch += [pltpu.VMEM((tail_n, d // 2), jnp.int32)]

    @functools.partial(
        pl.kernel,
        out_type=[jax.ShapeDtypeStruct((nt_rows, d), jnp.float32)] * 2,
        mesh=mesh,
        scratch_types=scratch,
    )
    def sc_pass(*refs):
        it = iter(refs)
        tbl_hbm = next(it)
        if with_ua:
            ua_hbm = next(it)
            next(it)
        src_hbm = next(it)
        dst_hbm = next(it)
        z_hbm = next(it)
        o0_hbm = next(it)
        o1_hbm = next(it)
        acc = next(it)
        rows_v = next(it)
        si = next(it)
        di = next(it)
        sem_i = next(it)
        sem_g = next(it)
        sem_s = next(it)
        sem_t = next(it)
        if with_ua:
            ua_v = next(it)
            sem_u = next(it)
        if tail_n:
            rows_t = next(it)
            si_t = next(it)
            di_t = next(it)
            if with_ua:
                ua_t = next(it)

        c = lax.axis_index("c")
        s = lax.axis_index("s")
        wid = s * NC + c
        row0 = s * rps
        base = e_off + wid * ppw
        ua_base = wid * ppw
        nfull = full

        def relu_add_row(rows, ua, r):
            for k in range(d // 32):
                u = ua[r, pl.ds(16 * k, LANES)]
                lo = lax.bitcast_convert_type(u << 16, jnp.float32)
                hi = lax.bitcast_convert_type(u & jnp.int32(-65536), jnp.float32)
                sl0 = pl.ds(16 * k, LANES)
                sl1 = pl.ds(d // 2 + 16 * k, LANES)
                rows[r, sl0] = jnp.maximum(rows[r, sl0] + lo, 0.0)
                rows[r, sl1] = jnp.maximum(rows[r, sl1] + hi, 0.0)

        def relu_add(b):
            def row_fn(r2, carry):
                for dr in range(2):
                    relu_add_row(rows_v.at[b], ua_v.at[b], r2 * 2 + dr)
                return carry
            lax.fori_loop(0, CH // 2, row_fn, 0)

        def pipeline():
            def fire_idx(g, b):
                off = base + g * CH
                pltpu.async_copy(src_hbm.at[pl.ds(off, CH)], si.at[b],
                                 sem_i.at[b])
                pltpu.async_copy(dst_hbm.at[pl.ds(off, CH)], di.at[b],
                                 sem_i.at[b])
                if with_ua:
                    uoff = ua_base + g * CH
                    pltpu.async_copy(ua_hbm.at[pl.ds(uoff, CH)], ua_v.at[b],
                                     sem_u.at[b])

            def wait_idx(b):
                pltpu.make_async_copy(src_hbm.at[pl.ds(0, CH)], si.at[b],
                                      sem_i.at[b]).wait()
                pltpu.make_async_copy(dst_hbm.at[pl.ds(0, CH)], di.at[b],
                                      sem_i.at[b]).wait()

            def fire_gather(b):
                pltpu.async_copy(tbl_hbm.at[si.at[b]], rows_v.at[b],
                                 sem_g.at[b])

            def wait_gather(b):
                pltpu.make_async_copy(tbl_hbm.at[si.at[b]], rows_v.at[b],
                                      sem_g.at[b]).wait()

            def wait_ua(b):
                pltpu.make_async_copy(ua_hbm.at[pl.ds(0, CH)], ua_v.at[b],
                                      sem_u.at[b]).wait()

            def fire_scatter(b):
                pltpu.async_copy(rows_v.at[b], acc.at[di.at[b]], sem_s.at[b],
                                 add=True)

            def wait_scatter(b):
                pltpu.make_async_copy(rows_v.at[b], acc.at[di.at[b]],
                                      sem_s.at[b]).wait()

            fire_idx(0, 0)
            fire_idx(1, 1)
            wait_idx(0)
            fire_gather(0)

            def group(i, carry):
                g0 = i * NBUF
                for db in range(NBUF):
                    g = g0 + db
                    b2 = (db + 2) % NBUF

                    @pl.when(jnp.logical_and(g + 2 >= NBUF, g + 2 < nfull))
                    def _():
                        wait_scatter(b2)

                    @pl.when(g + 2 < nfull)
                    def _():
                        fire_idx(g + 2, b2)

                    b1 = (db + 1) % NBUF

                    @pl.when(g + 1 < nfull)
                    def _():
                        wait_idx(b1)
                        fire_gather(b1)

                    if with_ua:
                        wait_ua(db)
                    wait_gather(db)
                    if with_ua:
                        relu_add(db)
                    fire_scatter(db)
                return carry

            lax.fori_loop(0, nfull // NBUF, group, 0)
            for b in range(NBUF):
                wait_scatter(b)

        pltpu.sync_copy(z_hbm, acc.at[pl.ds(row0, rps)])
        plsc.subcore_barrier()

        pipeline()

        if tail_n:
            off = base + full * CH
            pltpu.sync_copy(src_hbm.at[pl.ds(off, tail_n)], si_t)
            pltpu.sync_copy(dst_hbm.at[pl.ds(off, tail_n)], di_t)
            if with_ua:
                pltpu.sync_copy(
                    ua_hbm.at[pl.ds(ua_base + full * CH, tail_n)], ua_t)
            pltpu.async_copy(tbl_hbm.at[si_t], rows_t, sem_t).wait()

            if with_ua:
                def trow(r, carry):
                    relu_add_row(rows_t, ua_t, r)
                    return carry
                lax.fori_loop(0, tail_n, trow, 0)
            pltpu.sync_copy(rows_t, acc.at[di_t], add=True)

        plsc.subcore_barrier()

        @pl.when(c == 0)
        def _():
            pltpu.sync_copy(acc.at[pl.ds(row0, rps)], o0_hbm.at[pl.ds(row0, rps)])

        @pl.when(c == 1)
        def _():
            pltpu.sync_copy(acc.at[pl.ds(row0, rps)], o1_hbm.at[pl.ds(row0, rps)])

    return sc_pass


def kernel(x, up_index, up_attr, boundary_attr, boundary_index,
           W_msg_up, b_msg_up, W_up1, b_up1, W_up2, b_up2,
           W_b1, b_b1, W_b2, b_b2, W_comb, b_comb, eps1):
    n_cells, d = x.shape
    e_edges = up_attr.shape[0]
    eb = boundary_index.shape[1]

    rps = -(-(n_cells + 1) // NS)
    rps = -(-rps // 8) * 8
    nt_rows = rps * NS
    eb_pad = -(-eb // (NW * CH * NBUF)) * (NW * CH * NBUF)

    w_top = W_msg_up[:d]
    w_bot = W_msg_up[d:]

    src = up_index[0]
    dst = up_index[1]
    pad = eb_pad - eb
    bj = jnp.concatenate([boundary_index[0], jnp.zeros((pad,), jnp.int32)])
    trash = n_cells + (jnp.arange(pad, dtype=jnp.int32)
                       % jnp.int32(nt_rows - n_cells))
    bi = jnp.concatenate([boundary_index[1], trash])
    zrows = jnp.zeros((rps, d), jnp.float32)

    sc_bnd = _make_sc_pass(d, eb_pad, nt_rows, rps, with_ua=False)
    b0, b1 = sc_bnd(boundary_attr, bj, bi, zrows)

    eh = e_edges // 2
    xp = _tc_matmul(x, w_top, bn=1000)
    ua0 = _tc_matmul_bias(up_attr, w_bot, b_msg_up.reshape(1, d),
                          bn=2000, pack_u32=True, rows_m=eh, row_off=0)
    ua1 = _tc_matmul_bias(up_attr, w_bot, b_msg_up.reshape(1, d),
                          bn=2000, pack_u32=True, rows_m=eh, row_off=eh)

    sc_edge0 = _make_sc_pass(d, eh, nt_rows, rps, with_ua=True, e_off=0)
    sc_edge1 = _make_sc_pass(d, eh, nt_rows, rps, with_ua=True, e_off=eh)
    up0a, up1a = sc_edge0(xp, ua0, b0, src, dst, zrows)
    up0b, up1b = sc_edge1(xp, ua1, up0a, src, dst, zrows)

    scale = (1.0 + eps1).reshape(1, 1)
    out = _tc_final(x, up0a, up1a, up0b, up1b, b0, b1,
                    W_up1, b_up1.reshape(1, d), W_up2, b_up2.reshape(1, d),
                    W_b1, b_b1.reshape(1, d), W_b2, b_b2.reshape(1, d),
                    W_comb[:d], W_comb[d:], b_comb.reshape(1, d),
                    scale, bn=1000)
    return out


# speedup vs baseline: 2.7296x; 1.0042x over previous
import functools

import jax
import jax.numpy as jnp
import numpy as np
from jax import lax
from jax.experimental import pallas as pl
from jax.experimental.pallas import tpu as pltpu
from jax.experimental.pallas import tpu_sc as plsc

NC, NS, LANES = 2, 16, 16
NW = NC * NS
CH = 32
NBUF = 4


def _mm_bias_body(a_ref, w_ref, b_ref, o_ref):
    o_ref[...] = (
        jnp.dot(a_ref[...], w_ref[...], preferred_element_type=jnp.float32)
        + b_ref[...]
    ).astype(o_ref.dtype)


def _mm_bias_pack_body(a_ref, w_ref, b_ref, o_ref):
    v = (jnp.dot(a_ref[...], w_ref[...], preferred_element_type=jnp.float32)
         + b_ref[...])
    h = v.shape[1] // 2
    lo = lax.bitcast_convert_type(v[:, :h].astype(jnp.bfloat16), jnp.uint16)
    hi = lax.bitcast_convert_type(v[:, h:].astype(jnp.bfloat16), jnp.uint16)
    o_ref[...] = lo.astype(jnp.int32) | (hi.astype(jnp.int32) << 16)


def _mm_body(a_ref, w_ref, o_ref):
    o_ref[...] = jnp.dot(a_ref[...], w_ref[...], preferred_element_type=jnp.float32)


def _tc_matmul_bias(a, w, b, bn, pack_u32=False, rows_m=None, row_off=0):
    m, k = a.shape
    rows_m = m if rows_m is None else rows_m
    off_blocks = row_off // bn
    n = w.shape[1]
    body = _mm_bias_pack_body if pack_u32 else _mm_bias_body
    on = n // 2 if pack_u32 else n
    odt = jnp.int32 if pack_u32 else jnp.float32
    return pl.pallas_call(
        body,
        grid=(rows_m // bn,),
        in_specs=[
            pl.BlockSpec((bn, k), lambda i, o=off_blocks: (i + o, 0)),
            pl.BlockSpec((k, n), lambda i: (0, 0)),
            pl.BlockSpec((1, n), lambda i: (0, 0)),
        ],
        out_specs=pl.BlockSpec((bn, on), lambda i: (i, 0)),
        out_shape=jax.ShapeDtypeStruct((rows_m, on), odt),
    )(a, w, b)


def _tc_matmul(a, w, bn):
    m, k = a.shape
    n = w.shape[1]
    return pl.pallas_call(
        _mm_body,
        grid=(m // bn,),
        in_specs=[
            pl.BlockSpec((bn, k), lambda i: (i, 0)),
            pl.BlockSpec((k, n), lambda i: (0, 0)),
        ],
        out_specs=pl.BlockSpec((bn, n), lambda i: (i, 0)),
        out_shape=jax.ShapeDtypeStruct((m, n), jnp.float32),
    )(a, w)


def _final_body(x_ref, u0_ref, u1_ref, u2_ref, u3_ref, v0_ref, v1_ref,
                wu1_ref, bu1_ref, wu2_ref, bu2_ref,
                wb1_ref, bb1_ref, wb2_ref, bb2_ref,
                wc0_ref, wc1_ref, bc_ref, scale_ref, o_ref):
    scale = scale_ref[0, 0]
    xb = x_ref[...]
    h_up = (u0_ref[...] + u1_ref[...] + u2_ref[...] + u3_ref[...]
            + scale * xb)
    t = jnp.maximum(
        jnp.dot(h_up, wu1_ref[...], preferred_element_type=jnp.float32)
        + bu1_ref[...], 0.0)
    out_up = jnp.maximum(
        jnp.dot(t, wu2_ref[...], preferred_element_type=jnp.float32)
        + bu2_ref[...], 0.0)
    h_b = v0_ref[...] + v1_ref[...] + scale * xb
    t2 = jnp.maximum(
        jnp.dot(h_b, wb1_ref[...], preferred_element_type=jnp.float32)
        + bb1_ref[...], 0.0)
    out_b = jnp.maximum(
        jnp.dot(t2, wb2_ref[...], preferred_element_type=jnp.float32)
        + bb2_ref[...], 0.0)
    o_ref[...] = jnp.maximum(
        jnp.dot(out_up, wc0_ref[...], preferred_element_type=jnp.float32)
        + jnp.dot(out_b, wc1_ref[...], preferred_element_type=jnp.float32)
        + bc_ref[...], 0.0)


def _tc_final(x, u0, u1, u2, u3, v0, v1, wu1, bu1, wu2, bu2,
              wb1, bb1, wb2, bb2, wc0, wc1, bc, scale, bn):
    n_rows, d = x.shape
    mat = lambda: pl.BlockSpec((d, d), lambda i: (0, 0))
    vec = lambda: pl.BlockSpec((1, d), lambda i: (0, 0))
    rows = lambda: pl.BlockSpec((bn, d), lambda i: (i, 0))
    return pl.pallas_call(
        _final_body,
        grid=(n_rows // bn,),
        in_specs=[
            rows(), rows(), rows(), rows(), rows(), rows(), rows(),
            mat(), vec(), mat(), vec(),
            mat(), vec(), mat(), vec(),
            mat(), mat(), vec(),
            pl.BlockSpec(memory_space=pltpu.SMEM),
        ],
        out_specs=pl.BlockSpec((bn, d), lambda i: (i, 0)),
        out_shape=jax.ShapeDtypeStruct((n_rows, d), jnp.float32),
    )(x, u0, u1, u2, u3, v0, v1, wu1, bu1, wu2, bu2,
      wb1, bb1, wb2, bb2, wc0, wc1, bc, scale)


def _make_sc_pass(d, n_pairs, nt_rows, rps, with_ua, e_off=0, split=None):
    ppw = n_pairs // NW
    full = ppw // CH
    tail_n = ppw - full * CH
    assert full % NBUF == 0

    mesh = plsc.VectorSubcoreMesh(core_axis_name="c", subcore_axis_name="s")

    scratch = [
        pltpu.VMEM_SHARED((nt_rows, d), jnp.float32),
        pltpu.VMEM((NBUF, CH, d), jnp.float32),
        pltpu.VMEM((NBUF, CH), jnp.int32),
        pltpu.VMEM((NBUF, CH), jnp.int32),
        pltpu.SemaphoreType.DMA((NBUF,)),
        pltpu.SemaphoreType.DMA((NBUF,)),
        pltpu.SemaphoreType.DMA((NBUF,)),
        pltpu.SemaphoreType.DMA,
    ]
    if with_ua:
        scratch += [
            pltpu.VMEM((NBUF, CH, d // 2), jnp.int32),
            pltpu.SemaphoreType.DMA((NBUF,)),
        ]
    if tail_n:
        scratch += [
            pltpu.VMEM((tail_n, d), jnp.float32),
            pltpu.VMEM((tail_n,), jnp.int32),
            pltpu.VMEM((tail_n,), jnp.int32),
        ]
        if with_ua:
            scratch += [pltpu.VMEM((tail_n, d // 2), jnp.int32)]

    @functools.partial(
        pl.kernel,
        out_type=[jax.ShapeDtypeStruct((nt_rows, d), jnp.float32)] * 2,
        mesh=mesh,
        scratch_types=scratch,
    )
    def sc_pass(*refs):
        it = iter(refs)
        tbl_hbm = next(it)
        if with_ua:
            ua_hbm = next(it)
            next(it)
        src_hbm = next(it)
        dst_hbm = next(it)
        z_hbm = next(it)
        o0_hbm = next(it)
        o1_hbm = next(it)
        acc = next(it)
        rows_v = next(it)
        si = next(it)
        di = next(it)
        sem_i = next(it)
        sem_g = next(it)
        sem_s = next(it)
        sem_t = next(it)
        if with_ua:
            ua_v = next(it)
            sem_u = next(it)
        if tail_n:
            rows_t = next(it)
            si_t = next(it)
            di_t = next(it)
            if with_ua:
                ua_t = next(it)

        c = lax.axis_index("c")
        s = lax.axis_index("s")
        wid = s * NC + c
        row0 = s * rps
        base = e_off + wid * ppw
        ua_base = wid * ppw
        nfull = full
        if split is not None:
            n0, n1 = split
            assert n0 % NBUF == 0 and n1 % NBUF == 0
            assert (n0 + n1) * NS == (n_pairs // CH)

        def relu_add_row(rows, ua, r):
            for k in range(d // 32):
                u = ua[r, pl.ds(16 * k, LANES)]
                lo = lax.bitcast_convert_type(u << 16, jnp.float32)
                hi = lax.bitcast_convert_type(u & jnp.int32(-65536), jnp.float32)
                sl0 = pl.ds(16 * k, LANES)
                sl1 = pl.ds(d // 2 + 16 * k, LANES)
                rows[r, sl0] = jnp.maximum(rows[r, sl0] + lo, 0.0)
                rows[r, sl1] = jnp.maximum(rows[r, sl1] + hi, 0.0)

        def relu_add(b):
            def row_fn(r2, carry):
                for dr in range(2):
                    relu_add_row(rows_v.at[b], ua_v.at[b], r2 * 2 + dr)
                return carry
            lax.fori_loop(0, CH // 2, row_fn, 0)

        def pipeline(base, nfull):
            def fire_idx(g, b):
                off = base + g * CH
                pltpu.async_copy(src_hbm.at[pl.ds(off, CH)], si.at[b],
                                 sem_i.at[b])
                pltpu.async_copy(dst_hbm.at[pl.ds(off, CH)], di.at[b],
                                 sem_i.at[b])
                if with_ua:
                    uoff = ua_base + g * CH
                    pltpu.async_copy(ua_hbm.at[pl.ds(uoff, CH)], ua_v.at[b],
                                     sem_u.at[b])

            def wait_idx(b):
                pltpu.make_async_copy(src_hbm.at[pl.ds(0, CH)], si.at[b],
                                      sem_i.at[b]).wait()
                pltpu.make_async_copy(dst_hbm.at[pl.ds(0, CH)], di.at[b],
                                      sem_i.at[b]).wait()

            def fire_gather(b):
                pltpu.async_copy(tbl_hbm.at[si.at[b]], rows_v.at[b],
                                 sem_g.at[b])

            def wait_gather(b):
                pltpu.make_async_copy(tbl_hbm.at[si.at[b]], rows_v.at[b],
                                      sem_g.at[b]).wait()

            def wait_ua(b):
                pltpu.make_async_copy(ua_hbm.at[pl.ds(0, CH)], ua_v.at[b],
                                      sem_u.at[b]).wait()

            def fire_scatter(b):
                pltpu.async_copy(rows_v.at[b], acc.at[di.at[b]], sem_s.at[b],
                                 add=True)

            def wait_scatter(b):
                pltpu.make_async_copy(rows_v.at[b], acc.at[di.at[b]],
                                      sem_s.at[b]).wait()

            fire_idx(0, 0)
            fire_idx(1, 1)
            wait_idx(0)
            fire_gather(0)

            def group(i, carry):
                g0 = i * NBUF
                for db in range(NBUF):
                    g = g0 + db
                    b2 = (db + 2) % NBUF

                    @pl.when(jnp.logical_and(g + 2 >= NBUF, g + 2 < nfull))
                    def _():
                        wait_scatter(b2)

                    @pl.when(g + 2 < nfull)
                    def _():
                        fire_idx(g + 2, b2)

                    b1 = (db + 1) % NBUF

                    @pl.when(g + 1 < nfull)
                    def _():
                        wait_idx(b1)
                        fire_gather(b1)

                    if with_ua:
                        wait_ua(db)
                    wait_gather(db)
                    if with_ua:
                        relu_add(db)
                    fire_scatter(db)
                return carry

            lax.fori_loop(0, nfull // NBUF, group, 0)
            for b in range(NBUF):
                wait_scatter(b)

        pltpu.sync_copy(z_hbm, acc.at[pl.ds(row0, rps)])
        plsc.subcore_barrier()

        if split is None:
            pipeline(base, nfull)
        else:
            @pl.when(c == 0)
            def _():
                pipeline(s * (n0 * CH), n0)

            @pl.when(c == 1)
            def _():
                pipeline(NS * (n0 * CH) + s * (n1 * CH), n1)

        if tail_n:
            off = base + full * CH
            pltpu.sync_copy(src_hbm.at[pl.ds(off, tail_n)], si_t)
            pltpu.sync_copy(dst_hbm.at[pl.ds(off, tail_n)], di_t)
            if with_ua:
                pltpu.sync_copy(
                    ua_hbm.at[pl.ds(ua_base + full * CH, tail_n)], ua_t)
            pltpu.async_copy(tbl_hbm.at[si_t], rows_t, sem_t).wait()

            if with_ua:
                def trow(r, carry):
                    relu_add_row(rows_t, ua_t, r)
                    return carry
                lax.fori_loop(0, tail_n, trow, 0)
            pltpu.sync_copy(rows_t, acc.at[di_t], add=True)

        plsc.subcore_barrier()

        @pl.when(c == 0)
        def _():
            pltpu.sync_copy(acc.at[pl.ds(row0, rps)], o0_hbm.at[pl.ds(row0, rps)])

        @pl.when(c == 1)
        def _():
            pltpu.sync_copy(acc.at[pl.ds(row0, rps)], o1_hbm.at[pl.ds(row0, rps)])

    return sc_pass


def kernel(x, up_index, up_attr, boundary_attr, boundary_index,
           W_msg_up, b_msg_up, W_up1, b_up1, W_up2, b_up2,
           W_b1, b_b1, W_b2, b_b2, W_comb, b_comb, eps1):
    n_cells, d = x.shape
    e_edges = up_attr.shape[0]
    eb = boundary_index.shape[1]

    rps = -(-(n_cells + 1) // NS)
    rps = -(-rps // 8) * 8
    nt_rows = rps * NS
    eb_pad = -(-eb // (NW * CH * NBUF)) * (NW * CH * NBUF)

    w_top = W_msg_up[:d]
    w_bot = W_msg_up[d:]

    src = up_index[0]
    dst = up_index[1]
    pad = eb_pad - eb
    bj = jnp.concatenate([boundary_index[0], jnp.zeros((pad,), jnp.int32)])
    trash = n_cells + (jnp.arange(pad, dtype=jnp.int32)
                       % jnp.int32(nt_rows - n_cells))
    bi = jnp.concatenate([boundary_index[1], trash])
    zrows = jnp.zeros((rps, d), jnp.float32)

    sc_bnd = _make_sc_pass(d, eb_pad, nt_rows, rps, with_ua=False,
                           split=(136, 64))
    b0, b1 = sc_bnd(boundary_attr, bj, bi, zrows)

    eh = e_edges // 2
    xp = _tc_matmul(x, w_top, bn=1000)
    ua0 = _tc_matmul_bias(up_attr, w_bot, b_msg_up.reshape(1, d),
                          bn=2000, pack_u32=True, rows_m=eh, row_off=0)
    ua1 = _tc_matmul_bias(up_attr, w_bot, b_msg_up.reshape(1, d),
                          bn=2000, pack_u32=True, rows_m=eh, row_off=eh)

    sc_edge0 = _make_sc_pass(d, eh, nt_rows, rps, with_ua=True, e_off=0)
    sc_edge1 = _make_sc_pass(d, eh, nt_rows, rps, with_ua=True, e_off=eh)
    up0a, up1a = sc_edge0(xp, ua0, b0, src, dst, zrows)
    up0b, up1b = sc_edge1(xp, ua1, up0a, src, dst, zrows)

    scale = (1.0 + eps1).reshape(1, 1)
    out = _tc_final(x, up0a, up1a, up0b, up1b, b0, b1,
                    W_up1, b_up1.reshape(1, d), W_up2, b_up2.reshape(1, d),
                    W_b1, b_b1.reshape(1, d), W_b2, b_b2.reshape(1, d),
                    W_comb[:d], W_comb[d:], b_comb.reshape(1, d),
                    scale, bn=1000)
    return out
